# Initial kernel scaffold; baseline (speedup 1.0000x reference)
import jax
import jax.numpy as jnp
from jax.experimental import pallas as pl


def kernel(xyz, features, W, b, gamma, beta, running_mean, running_var):
    raise NotImplementedError("write your pallas kernel here")


# speedup vs baseline: 1.0012x; 1.0012x over previous
import jax
import jax.numpy as jnp
from jax.experimental import pallas as pl
from jax.experimental.pallas import tpu as pltpu

NPOINT = 2048
K = 16
IN_DIM = 64
OUT_DIM = 128
B = 4
N = 8192


def _mlp_body(x_ref, wt_ref, b_ref, g_ref, bt_ref, mu_ref, var_ref, o_ref):
    h = jnp.dot(x_ref[...], wt_ref[...], preferred_element_type=jnp.float32)
    h = h + b_ref[...]
    h = (h - mu_ref[...]) / jnp.sqrt(var_ref[...] + 1e-5) * g_ref[...] + bt_ref[...]
    o_ref[...] = jax.nn.relu(h)


def _mlp(features, W, b, gamma, beta, mean, var):
    x = features.reshape(B * N, IN_DIM)
    wt = W.T
    vec = lambda v: v.reshape(1, OUT_DIM)
    grid = 16
    rows = (B * N) // grid
    out = pl.pallas_call(
        _mlp_body,
        grid=(grid,),
        in_specs=[
            pl.BlockSpec((rows, IN_DIM), lambda i: (i, 0)),
            pl.BlockSpec((IN_DIM, OUT_DIM), lambda i: (0, 0)),
            pl.BlockSpec((1, OUT_DIM), lambda i: (0, 0)),
            pl.BlockSpec((1, OUT_DIM), lambda i: (0, 0)),
            pl.BlockSpec((1, OUT_DIM), lambda i: (0, 0)),
            pl.BlockSpec((1, OUT_DIM), lambda i: (0, 0)),
            pl.BlockSpec((1, OUT_DIM), lambda i: (0, 0)),
        ],
        out_specs=pl.BlockSpec((rows, OUT_DIM), lambda i: (i, 0)),
        out_shape=jax.ShapeDtypeStruct((B * N, OUT_DIM), jnp.float32),
    )(x, wt, vec(b), vec(gamma), vec(beta), vec(mean), vec(var))
    return out.reshape(B, N, OUT_DIM)


def _fps(xyz, npoint):
    b_, n_, _ = xyz.shape
    farthest = jax.random.randint(jax.random.key(1), (b_,), 0, n_).astype(jnp.int32)
    distance = jnp.full((b_, n_), 1e10, dtype=jnp.float32)
    centroids = jnp.zeros((b_, npoint), dtype=jnp.int32)

    def body(i, carry):
        centroids, distance, farthest = carry
        centroids = centroids.at[:, i].set(farthest)
        centroid = jnp.take_along_axis(xyz, farthest[:, None, None], axis=1)
        dist = jnp.sum((xyz - centroid) ** 2, axis=-1)
        distance = jnp.minimum(distance, dist)
        farthest = jnp.argmax(distance, axis=-1).astype(jnp.int32)
        return (centroids, distance, farthest)

    centroids, _, _ = jax.lax.fori_loop(0, npoint, body, (centroids, distance, farthest))
    return centroids


def kernel(xyz, features, W, b, gamma, beta, running_mean, running_var):
    fps_idx = _fps(xyz, NPOINT)
    new_xyz = jnp.take_along_axis(xyz, fps_idx[:, :, None], axis=1)
    aa = jnp.sum(new_xyz * new_xyz, axis=-1)[:, :, None]
    bb = jnp.sum(xyz * xyz, axis=-1)[:, None, :]
    ab = jnp.einsum('bmd,bnd->bmn', new_xyz, xyz)
    dists = aa + bb - 2.0 * ab
    _, idx = jax.lax.top_k(-dists, K)
    h = _mlp(features, W, b, gamma, beta, running_mean, running_var)
    grouped = jnp.take_along_axis(h, idx.reshape(B, -1)[:, :, None], axis=1)
    grouped = grouped.reshape(B, NPOINT, K, OUT_DIM)
    new_features = jnp.max(grouped, axis=2)
    return (new_xyz, new_features)


# speedup vs baseline: 10.5167x; 10.5046x over previous
import functools

import jax
import jax.numpy as jnp
from jax import lax
from jax.experimental import pallas as pl
from jax.experimental.pallas import tpu as pltpu
from jax.experimental.pallas import tpu_sc as plsc

NPOINT = 2048
K = 16
IN_DIM = 64
OUT_DIM = 128
B = 4
N = 8192

_BIG = 3.4e38
_NEG = -3.4e38


def _fps_body(start_ref, xyz_ref, nx_ref, dist_ref):
    pidx = (lax.broadcasted_iota(jnp.int32, (8, 1024), 0) * 1024
            + lax.broadcasted_iota(jnp.int32, (8, 1024), 1))
    slotidx = (lax.broadcasted_iota(jnp.int32, (8, 256), 1) * 8
               + lax.broadcasted_iota(jnp.int32, (8, 256), 0))
    for b in range(B):
        dist_ref[b] = jnp.full((8, 1024), 1e10, jnp.float32)

    def step(i, far):
        newfar = []
        for b in range(B):
            fb = far[b]
            x = xyz_ref[b, 0]
            y = xyz_ref[b, 1]
            z = xyz_ref[b, 2]
            mask = pidx == fb
            cx = jnp.max(jnp.where(mask, x, _NEG))
            cy = jnp.max(jnp.where(mask, y, _NEG))
            cz = jnp.max(jnp.where(mask, z, _NEG))
            smask = slotidx == i
            nx_ref[b, 0] = jnp.where(smask, cx, nx_ref[b, 0])
            nx_ref[b, 1] = jnp.where(smask, cy, nx_ref[b, 1])
            nx_ref[b, 2] = jnp.where(smask, cz, nx_ref[b, 2])
            dx = x - cx
            dy = y - cy
            dz = z - cz
            dd = dx * dx + dy * dy + dz * dz
            dnew = jnp.minimum(dist_ref[b], dd)
            dist_ref[b] = dnew
            m = jnp.max(dnew)
            sel = jnp.where(dnew == m, pidx, jnp.int32(2**30))
            newfar.append(jnp.min(sel))
        return tuple(newfar)

    lax.fori_loop(0, NPOINT, step, tuple(start_ref[b] for b in range(B)))


def _fps(xyz_p, start):
    return pl.pallas_call(
        _fps_body,
        in_specs=[
            pl.BlockSpec(memory_space=pltpu.SMEM),
            pl.BlockSpec(memory_space=pltpu.VMEM),
        ],
        out_specs=pl.BlockSpec(memory_space=pltpu.VMEM),
        out_shape=jax.ShapeDtypeStruct((B, 3, 8, 256), jnp.float32),
        scratch_shapes=[pltpu.VMEM((B, 8, 1024), jnp.float32)],
    )(start, xyz_p)


def _knn_body(nxq_ref, xyzt_ref, out_ref, d_ref):
    b = pl.program_id(0)
    q = nxq_ref[0]
    p = xyzt_ref[0]
    qx, qy, qz = q[:, 0:1], q[:, 1:2], q[:, 2:3]
    px, py, pz = p[0:1, :], p[1:2, :], p[2:3, :]
    aa = qx * qx + qy * qy + qz * qz
    bb = px * px + py * py + pz * pz
    ab = jnp.dot(q.astype(jnp.bfloat16), p.astype(jnp.bfloat16),
                 preferred_element_type=jnp.float32)
    d_ref[...] = aa + bb - 2.0 * ab
    iota_col = lax.broadcasted_iota(jnp.int32, (256, N), 1)
    cols = []
    for _ in range(K):
        dk = d_ref[...]
        m = jnp.min(dk, axis=1, keepdims=True)
        sel = jnp.where(dk == m, iota_col, jnp.int32(2**30))
        idx = jnp.min(sel, axis=1, keepdims=True)
        d_ref[...] = jnp.where(iota_col == idx, _BIG, dk)
        cols.append(idx)
    out_ref[0, 0] = jnp.concatenate(cols, axis=1) + b * N


def _knn(new_xyz, xyz_t):
    return pl.pallas_call(
        _knn_body,
        grid=(B, 8),
        in_specs=[
            pl.BlockSpec((1, 256, 3), lambda b, t: (b, t, 0)),
            pl.BlockSpec((1, 3, N), lambda b, t: (b, 0, 0)),
        ],
        out_specs=pl.BlockSpec((1, 1, 256, K), lambda b, t: (b, t, 0, 0)),
        out_shape=jax.ShapeDtypeStruct((B, 8, 256, K), jnp.int32),
        scratch_shapes=[pltpu.VMEM((256, N), jnp.float32)],
    )(new_xyz, xyz_t)


def _mlp_body(x_ref, wt_ref, b_ref, g_ref, bt_ref, mu_ref, var_ref, o_ref):
    h = jnp.dot(x_ref[...], wt_ref[...], preferred_element_type=jnp.float32)
    h = h + b_ref[...]
    h = (h - mu_ref[...]) / jnp.sqrt(var_ref[...] + 1e-5) * g_ref[...] + bt_ref[...]
    o_ref[...] = jax.nn.relu(h)


def _mlp(features, W, b, gamma, beta, mean, var):
    x = features.reshape(B * N, IN_DIM)
    wt = W.T
    vec = lambda v: v.reshape(1, OUT_DIM)
    grid = 16
    rows = (B * N) // grid
    vspec = pl.BlockSpec((1, OUT_DIM), lambda i: (0, 0))
    return pl.pallas_call(
        _mlp_body,
        grid=(grid,),
        in_specs=[
            pl.BlockSpec((rows, IN_DIM), lambda i: (i, 0)),
            pl.BlockSpec((IN_DIM, OUT_DIM), lambda i: (0, 0)),
            vspec, vspec, vspec, vspec, vspec,
        ],
        out_specs=pl.BlockSpec((rows, OUT_DIM), lambda i: (i, 0)),
        out_shape=jax.ShapeDtypeStruct((B * N, OUT_DIM), jnp.float32),
    )(x, wt, vec(b), vec(gamma), vec(beta), vec(mean), vec(var))


_NC, _NS = 2, 16
_NW = _NC * _NS
_QTOT = B * NPOINT
_QW = _QTOT // _NW
_QC = 8
_NCHUNK = _QW // _QC


def _group_body(h_hbm, idx_hbm, out_hbm, idx_v, rows_v, out_v, sem):
    wid = lax.axis_index("s") * _NC + lax.axis_index("c")
    qbase = wid * _QW

    def chunk(c, _):
        pltpu.sync_copy(idx_hbm.at[pl.ds(qbase * K + c * (_QC * K), _QC * K)],
                        idx_v)
        pltpu.async_copy(h_hbm.at[idx_v], rows_v, sem).wait()

        def per_q(qi, _):
            base = qi * K
            for j in range(OUT_DIM // 16):
                acc = rows_v[base, pl.ds(j * 16, 16)]
                for r in range(1, K):
                    acc = jnp.maximum(acc, rows_v[base + r, pl.ds(j * 16, 16)])
                out_v[qi, pl.ds(j * 16, 16)] = acc
            return 0

        lax.fori_loop(0, _QC, per_q, 0)
        pltpu.sync_copy(out_v, out_hbm.at[pl.ds(qbase + c * _QC, _QC)])
        return 0

    lax.fori_loop(0, _NCHUNK, chunk, 0)


def _group_max(h_flat, idx_flat):
    mesh = plsc.VectorSubcoreMesh(core_axis_name="c", subcore_axis_name="s",
                                  num_cores=_NC)
    f = functools.partial(
        pl.kernel,
        out_type=jax.ShapeDtypeStruct((_QTOT, OUT_DIM), jnp.float32),
        mesh=mesh,
        scratch_types=[
            pltpu.VMEM((_QC * K,), jnp.int32),
            pltpu.VMEM((_QC * K, OUT_DIM), jnp.float32),
            pltpu.VMEM((_QC, OUT_DIM), jnp.float32),
            pltpu.SemaphoreType.DMA,
        ],
    )(_group_body)
    return f(h_flat, idx_flat)


def kernel(xyz, features, W, b, gamma, beta, running_mean, running_var):
    start = jax.random.randint(jax.random.key(1), (B,), 0, N).astype(jnp.int32)
    xyz_t = jnp.transpose(xyz, (0, 2, 1))
    xyz_p = xyz_t.reshape(B, 3, 8, 1024)

    nx_cr = _fps(xyz_p, start)
    new_xyz = jnp.transpose(nx_cr, (0, 1, 3, 2)).reshape(B, 3, NPOINT)
    new_xyz = jnp.transpose(new_xyz, (0, 2, 1))

    idx = _knn(new_xyz, xyz_t)
    idx_flat = idx.reshape(_QTOT * K)

    h_flat = _mlp(features, W, b, gamma, beta, running_mean, running_var)

    new_features = _group_max(h_flat, idx_flat).reshape(B, NPOINT, OUT_DIM)
    return (new_xyz, new_features)


# speedup vs baseline: 19.7044x; 1.8736x over previous
import functools

import jax
import jax.numpy as jnp
from jax import lax
from jax.experimental import pallas as pl
from jax.experimental.pallas import tpu as pltpu
from jax.experimental.pallas import tpu_sc as plsc

NPOINT = 2048
K = 16
IN_DIM = 64
OUT_DIM = 128
B = 4
N = 8192

_BIG = 3.4e38
_NEG = -3.4e38


def _fps_body(start_ref, xyz_ref, idx_ref, d0, d1, d2, d3):
    dref = (d0, d1, d2, d3)
    pidx = (lax.broadcasted_iota(jnp.int32, (8, 1024), 0) * 1024
            + lax.broadcasted_iota(jnp.int32, (8, 1024), 1))
    slotidx = (lax.broadcasted_iota(jnp.int32, (8, 256), 1) * 8
               + lax.broadcasted_iota(jnp.int32, (8, 256), 0))
    for b in range(B):
        dref[b][...] = jnp.full((8, 1024), 1e10, jnp.float32)
    acc0 = [jnp.zeros((8, 256), jnp.int32) for _ in range(B)]
    far0 = [jnp.full((1, 1), start_ref[b], jnp.int32) for b in range(B)]

    def step(i, carry):
        far, acc = carry
        nfar, nacc = [], []
        for b in range(B):
            fb = far[b]
            x = xyz_ref[b, 0]
            y = xyz_ref[b, 1]
            z = xyz_ref[b, 2]
            mask = pidx == fb
            red = lambda a: jnp.max(jnp.max(a, axis=0, keepdims=True),
                                    axis=1, keepdims=True)
            cx = red(jnp.where(mask, x, _NEG))
            cy = red(jnp.where(mask, y, _NEG))
            cz = red(jnp.where(mask, z, _NEG))
            nacc.append(jnp.where(slotidx == i, fb, acc[b]))
            dx = x - cx
            dy = y - cy
            dz = z - cz
            dd = dx * dx + dy * dy + dz * dz
            dnew = jnp.minimum(dref[b][...], dd)
            dref[b][...] = dnew
            m = red(dnew)
            sel = jnp.where(dnew == m, pidx, jnp.int32(2**30))
            nfar.append(jnp.min(jnp.min(sel, axis=0, keepdims=True),
                                axis=1, keepdims=True))
        return tuple(nfar), tuple(nacc)

    _, acc = lax.fori_loop(0, NPOINT, step, (tuple(far0), tuple(acc0)))
    for b in range(B):
        idx_ref[b] = acc[b] + b * N


def _fps(xyz_p, start):
    return pl.pallas_call(
        _fps_body,
        in_specs=[
            pl.BlockSpec(memory_space=pltpu.SMEM),
            pl.BlockSpec(memory_space=pltpu.VMEM),
        ],
        out_specs=pl.BlockSpec(memory_space=pltpu.VMEM),
        out_shape=jax.ShapeDtypeStruct((B, 8, 256), jnp.int32),
        scratch_shapes=[pltpu.VMEM((8, 1024), jnp.float32) for _ in range(B)],
    )(start, xyz_p)


def _xyzgather_body(tab_hbm, idx_hbm, out_hbm, idx_v, rows_v, sem):
    wid = lax.axis_index("s") * _NC + lax.axis_index("c")
    base = wid * (NPOINT * B // _NW)
    for c in range(NPOINT * B // _NW // 128):
        pltpu.sync_copy(idx_hbm.at[pl.ds(base + c * 128, 128)], idx_v)
        pltpu.async_copy(tab_hbm.at[idx_v], rows_v, sem).wait()
        pltpu.sync_copy(rows_v, out_hbm.at[pl.ds(base + c * 128, 128)])


def _gather_xyz(xyz_pad, idx_flat):
    mesh = plsc.VectorSubcoreMesh(core_axis_name="c", subcore_axis_name="s",
                                  num_cores=_NC)
    f = functools.partial(
        pl.kernel,
        out_type=jax.ShapeDtypeStruct((B * NPOINT, 128), jnp.float32),
        mesh=mesh,
        scratch_types=[
            pltpu.VMEM((128,), jnp.int32),
            pltpu.VMEM((128, 128), jnp.float32),
            pltpu.SemaphoreType.DMA,
        ],
    )(_xyzgather_body)
    return f(xyz_pad, idx_flat)


def _knn_body(nxq_ref, xyzt_ref, out_ref, d_ref):
    b = pl.program_id(0)
    q = nxq_ref[0]
    p = xyzt_ref[0]
    qx, qy, qz = q[:, 0:1], q[:, 1:2], q[:, 2:3]
    px, py, pz = p[0:1, :], p[1:2, :], p[2:3, :]
    aa = qx * qx + qy * qy + qz * qz
    bb = px * px + py * py + pz * pz
    ab = jnp.dot(q.astype(jnp.bfloat16), p.astype(jnp.bfloat16),
                 preferred_element_type=jnp.float32)
    d_ref[...] = aa + bb - 2.0 * ab
    iota_col = lax.broadcasted_iota(jnp.int32, (256, N), 1)
    cols = []
    for _ in range(K):
        dk = d_ref[...]
        m = jnp.min(dk, axis=1, keepdims=True)
        sel = jnp.where(dk == m, iota_col, jnp.int32(2**30))
        idx = jnp.min(sel, axis=1, keepdims=True)
        d_ref[...] = jnp.where(iota_col == idx, _BIG, dk)
        cols.append(idx)
    out_ref[0, 0] = jnp.concatenate(cols, axis=1) + b * N


def _knn(new_xyz, xyz_t):
    return pl.pallas_call(
        _knn_body,
        grid=(B, 8),
        in_specs=[
            pl.BlockSpec((1, 256, 3), lambda b, t: (b, t, 0)),
            pl.BlockSpec((1, 3, N), lambda b, t: (b, 0, 0)),
        ],
        out_specs=pl.BlockSpec((1, 1, 256, K), lambda b, t: (b, t, 0, 0)),
        out_shape=jax.ShapeDtypeStruct((B, 8, 256, K), jnp.int32),
        scratch_shapes=[pltpu.VMEM((256, N), jnp.float32)],
    )(new_xyz, xyz_t)


def _mlp_body(x_ref, wt_ref, b_ref, g_ref, bt_ref, mu_ref, var_ref, o_ref):
    h = jnp.dot(x_ref[...], wt_ref[...], preferred_element_type=jnp.float32)
    h = h + b_ref[...]
    h = (h - mu_ref[...]) / jnp.sqrt(var_ref[...] + 1e-5) * g_ref[...] + bt_ref[...]
    o_ref[...] = jax.nn.relu(h)


def _mlp(features, W, b, gamma, beta, mean, var):
    x = features.reshape(B * N, IN_DIM)
    wt = W.T
    vec = lambda v: v.reshape(1, OUT_DIM)
    grid = 16
    rows = (B * N) // grid
    vspec = pl.BlockSpec((1, OUT_DIM), lambda i: (0, 0))
    return pl.pallas_call(
        _mlp_body,
        grid=(grid,),
        in_specs=[
            pl.BlockSpec((rows, IN_DIM), lambda i: (i, 0)),
            pl.BlockSpec((IN_DIM, OUT_DIM), lambda i: (0, 0)),
            vspec, vspec, vspec, vspec, vspec,
        ],
        out_specs=pl.BlockSpec((rows, OUT_DIM), lambda i: (i, 0)),
        out_shape=jax.ShapeDtypeStruct((B * N, OUT_DIM), jnp.float32),
    )(x, wt, vec(b), vec(gamma), vec(beta), vec(mean), vec(var))


_NC, _NS = 2, 16
_NW = _NC * _NS
_QTOT = B * NPOINT
_QW = _QTOT // _NW
_QC = 8
_NCHUNK = _QW // _QC


def _group_body(h_hbm, idx_hbm, out_hbm, idx_v, rows_v, out_v, sem):
    wid = lax.axis_index("s") * _NC + lax.axis_index("c")
    qbase = wid * _QW

    def chunk(c, _):
        pltpu.sync_copy(idx_hbm.at[pl.ds(qbase * K + c * (_QC * K), _QC * K)],
                        idx_v)
        pltpu.async_copy(h_hbm.at[idx_v], rows_v, sem).wait()

        def per_q(qi, _):
            base = qi * K
            for j in range(OUT_DIM // 16):
                acc = rows_v[base, pl.ds(j * 16, 16)]
                for r in range(1, K):
                    acc = jnp.maximum(acc, rows_v[base + r, pl.ds(j * 16, 16)])
                out_v[qi, pl.ds(j * 16, 16)] = acc
            return 0

        lax.fori_loop(0, _QC, per_q, 0)
        pltpu.sync_copy(out_v, out_hbm.at[pl.ds(qbase + c * _QC, _QC)])
        return 0

    lax.fori_loop(0, _NCHUNK, chunk, 0)


def _group_max(h_flat, idx_flat):
    mesh = plsc.VectorSubcoreMesh(core_axis_name="c", subcore_axis_name="s",
                                  num_cores=_NC)
    f = functools.partial(
        pl.kernel,
        out_type=jax.ShapeDtypeStruct((_QTOT, OUT_DIM), jnp.float32),
        mesh=mesh,
        scratch_types=[
            pltpu.VMEM((_QC * K,), jnp.int32),
            pltpu.VMEM((_QC * K, OUT_DIM), jnp.float32),
            pltpu.VMEM((_QC, OUT_DIM), jnp.float32),
            pltpu.SemaphoreType.DMA,
        ],
    )(_group_body)
    return f(h_flat, idx_flat)


def kernel(xyz, features, W, b, gamma, beta, running_mean, running_var):
    start = jax.random.randint(jax.random.key(1), (B,), 0, N).astype(jnp.int32)
    xyz_t = jnp.transpose(xyz, (0, 2, 1))
    xyz_p = xyz_t.reshape(B, 3, 8, 1024)

    idx_cr = _fps(xyz_p, start)
    fps_flat = jnp.transpose(idx_cr, (0, 2, 1)).reshape(B * NPOINT)
    xyz_pad = jnp.pad(xyz.reshape(B * N, 3), ((0, 0), (0, 125)))
    nx_rows = _gather_xyz(xyz_pad, fps_flat)
    new_xyz = nx_rows[:, :3].reshape(B, NPOINT, 3)

    idx = _knn(new_xyz, xyz_t)
    idx_flat = idx.reshape(_QTOT * K)

    h_flat = _mlp(features, W, b, gamma, beta, running_mean, running_var)

    new_features = _group_max(h_flat, idx_flat).reshape(B, NPOINT, OUT_DIM)
    return (new_xyz, new_features)
